# baseline (speedup 1.0000x reference)
import functools

import jax
import jax.numpy as jnp
from jax import lax
from jax.experimental import pallas as pl
from jax.experimental.pallas import tpu as pltpu
from jax.experimental.pallas import tpu_sc as plsc

_K = 16
_N = 20000
_NPAD = 20480
_BQ = 128
_CK = 2048
_SEG = 128
_NS = _NPAD // _SEG
_PADVAL = 1.0e18
_INF = 3.0e38
_BIG = 1.0e9

_NTILES = 32
_QPT = _NPAD // _NTILES
_G = 64
_NGR = _QPT // _G
_POOL = 2048


def _tc_block(q_ref, kT_ref, sm_ref, t0_ref):
    q = q_ref[...]
    qx = q[:, 0:1]
    qy = q[:, 1:2]
    qz = q[:, 2:3]
    q2 = qx * qx + qy * qy + qz * qz
    qb = q.astype(jnp.bfloat16)

    sms = []
    for c in range(_NPAD // _CK):
        kTc = kT_ref[:, c * _CK:(c + 1) * _CK]
        kx = kTc[0:1, :]
        ky = kTc[1:2, :]
        kz = kTc[2:3, :]
        k2 = kx * kx + ky * ky + kz * kz
        dot = lax.dot_general(qb, kTc.astype(jnp.bfloat16),
                              (((1,), (0,)), ((), ())),
                              preferred_element_type=jnp.float32)
        sq = jnp.maximum((q2 + k2) - 2.0 * dot, 0.0)
        sms.append(jnp.min(sq.reshape(_BQ, _CK // _SEG, _SEG), axis=2))
    sm = jnp.concatenate(sms, axis=1)
    sm_ref[...] = sm

    iota_s = jax.lax.broadcasted_iota(jnp.int32, (_BQ, _NS), 1).astype(jnp.float32)
    mv = jnp.min(sm, axis=1, keepdims=True)
    for _ in range(_K - 1):
        first = jnp.min(jnp.where(sm == mv, iota_s, _BIG), axis=1, keepdims=True)
        sm = jnp.where(iota_s == first, _INF, sm)
        mv = jnp.min(sm, axis=1, keepdims=True)
    t0_ref[...] = jnp.broadcast_to(mv, (_BQ, 8))


def _sc_select(sm_hbm, t0_hbm, kx_hbm, ky_hbm, kz_hbm, k2_hbm, out_hbm,
               kxv, kyv, kzv, k2v, smb, t0b, outb, segbuf, poold, pooli):
    wid = lax.axis_index("s") * 2 + lax.axis_index("c")
    base = wid * _QPT
    pltpu.sync_copy(kx_hbm, kxv)
    pltpu.sync_copy(ky_hbm, kyv)
    pltpu.sync_copy(kz_hbm, kzv)
    pltpu.sync_copy(k2_hbm, k2v)

    nq = jnp.maximum(jnp.minimum(_N - base, _QPT), 0)
    iota = jax.lax.iota(jnp.int32, 16)

    def group_body(g, _):
        q0 = base + g * _G
        pltpu.sync_copy(sm_hbm.at[pl.ds(q0 * _NS, _G * _NS)], smb)
        pltpu.sync_copy(t0_hbm.at[pl.ds(q0 * 8, _G * 8)], t0b)
        nql = jnp.minimum(_G, nq - g * _G)

        def query_body(ql, _):
            t0v = plsc.load_gather(t0b, [jnp.full((16,), ql * 8, jnp.int32)])
            t0m = t0v * 1.00001 + 1e-4
            qg = jnp.full((16,), base + g * _G + ql, jnp.int32)
            qxv = plsc.load_gather(kxv, [qg])
            qyv = plsc.load_gather(kyv, [qg])
            qzv = plsc.load_gather(kzv, [qg])
            q2v = plsc.load_gather(k2v, [qg])

            scnt = jnp.int32(0)
            for c in range(_NS // 16):
                m = plsc.load_gather(
                    smb, [jnp.full((16,), ql * _NS + c * 16, jnp.int32) + iota])
                msk = m <= t0m
                cs = plsc.cumsum(msk.astype(jnp.int32))
                plsc.store_scatter(segbuf, [scnt + cs - 1], iota + c * 16,
                                   mask=msk)
                scnt = scnt + jnp.max(cs)

            def seg_body(i, pcnt):
                sv = plsc.load_gather(segbuf, [jnp.full((16,), i, jnp.int32)])
                for sub in range(_SEG // 16):
                    kidx = sv * _SEG + sub * 16 + iota
                    kxg = plsc.load_gather(kxv, [kidx])
                    kyg = plsc.load_gather(kyv, [kidx])
                    kzg = plsc.load_gather(kzv, [kidx])
                    k2g = plsc.load_gather(k2v, [kidx])
                    dot = (qxv * kxg + qyv * kyg) + qzv * kzg
                    sq = jnp.maximum((q2v + k2g) - 2.0 * dot, 0.0)
                    msk = sq <= t0m
                    nhit = jnp.max(plsc.all_reduce_population_count(msk))

                    @pl.when(nhit > 0)
                    def _():
                        cs = plsc.cumsum(msk.astype(jnp.int32))
                        pos = jnp.minimum(pcnt + cs - 1, _POOL + 15)
                        plsc.store_scatter(poold, [pos], sq, mask=msk)
                        plsc.store_scatter(pooli, [pos], kidx, mask=msk)

                    pcnt = pcnt + nhit
                return pcnt

            pcnt = lax.fori_loop(0, scnt, seg_body, jnp.int32(0))
            pc = jnp.minimum(pcnt, _POOL)
            plsc.store_scatter(poold, [jnp.full((16,), pc, jnp.int32) + iota],
                               jnp.full((16,), _INF, jnp.float32))

            bd, bi = plsc.sort_key_val(poold[pl.ds(0, 16)], pooli[pl.ds(0, 16)])

            def mrg(c, carry):
                bd, bi = carry
                cidx = jnp.full((16,), c * 16, jnp.int32) + iota
                cd, ci = plsc.sort_key_val(plsc.load_gather(poold, [cidx]),
                                           plsc.load_gather(pooli, [cidx]),
                                           descending=True)
                sel = cd < bd
                md = jnp.where(sel, cd, bd)
                mi = jnp.where(sel, ci, bi)
                return tuple(plsc.sort_key_val(md, mi))

            bd, bi = lax.fori_loop(1, (pc + 15) // 16, mrg, (bd, bi))
            plsc.store_scatter(outb, [jnp.full((16,), ql * _K, jnp.int32) + iota],
                               bi.astype(jnp.float32))
            return 0

        lax.fori_loop(0, nql, query_body, 0)
        pltpu.sync_copy(outb, out_hbm.at[pl.ds(q0 * _K, _G * _K)])
        return 0

    lax.fori_loop(0, _NGR, group_body, 0)


@jax.jit
def kernel(barycenters):
    n = barycenters.shape[0]
    pad = jnp.full((_NPAD - n, 3), _PADVAL, dtype=jnp.float32)
    bpad = jnp.concatenate([barycenters, pad], axis=0)
    kT = bpad.T

    sm, t0 = pl.pallas_call(
        _tc_block,
        grid=(_NPAD // _BQ,),
        in_specs=[
            pl.BlockSpec((_BQ, 3), lambda i: (i, 0)),
            pl.BlockSpec((3, _NPAD), lambda i: (0, 0)),
        ],
        out_specs=[
            pl.BlockSpec((_BQ, _NS), lambda i: (i, 0)),
            pl.BlockSpec((_BQ, 8), lambda i: (i, 0)),
        ],
        out_shape=[
            jax.ShapeDtypeStruct((_NPAD, _NS), jnp.float32),
            jax.ShapeDtypeStruct((_NPAD, 8), jnp.float32),
        ],
    )(bpad, kT)

    kx = bpad[:, 0]
    ky = bpad[:, 1]
    kz = bpad[:, 2]
    k2 = kx * kx + ky * ky + kz * kz

    def bf16_round(x):
        b = lax.bitcast_convert_type(x, jnp.int32)
        r = (b + 0x7FFF + ((b >> 16) & 1)) & jnp.int32(-65536)
        return lax.bitcast_convert_type(r, jnp.float32)

    kxt = bf16_round(kx)
    kyt = bf16_round(ky)
    kzt = bf16_round(kz)

    sc = pl.kernel(
        _sc_select,
        out_type=jax.ShapeDtypeStruct((_NPAD * _K,), jnp.float32),
        mesh=plsc.VectorSubcoreMesh(core_axis_name="c", subcore_axis_name="s",
                                    num_cores=2, num_subcores=16),
        compiler_params=pltpu.CompilerParams(needs_layout_passes=False),
        scratch_types=[
            pltpu.VMEM((_NPAD,), jnp.float32),
            pltpu.VMEM((_NPAD,), jnp.float32),
            pltpu.VMEM((_NPAD,), jnp.float32),
            pltpu.VMEM((_NPAD,), jnp.float32),
            pltpu.VMEM((_G * _NS,), jnp.float32),
            pltpu.VMEM((_G * 8,), jnp.float32),
            pltpu.VMEM((_G * _K,), jnp.float32),
            pltpu.VMEM((_NS + 16,), jnp.int32),
            pltpu.VMEM((_POOL + 16,), jnp.float32),
            pltpu.VMEM((_POOL + 16,), jnp.int32),
        ],
    )
    outf = sc(sm.reshape(-1), t0.reshape(-1), kxt, kyt, kzt, k2)
    return outf.reshape(_NPAD, _K)[:n]


# speedup vs baseline: 1.6961x; 1.6961x over previous
import functools

import jax
import jax.numpy as jnp
from jax import lax
from jax.experimental import pallas as pl
from jax.experimental.pallas import tpu as pltpu
from jax.experimental.pallas import tpu_sc as plsc

_K = 16
_N = 20000
_NPAD = 20480
_BQ = 128
_CK = 2048
_SEG = 128
_NS = _NPAD // _SEG
_PADVAL = 1.0e18
_INF = 3.0e38
_BIG = 1.0e9

_NTILES = 32
_NHALF = 2
_HQ = _NPAD // _NHALF
_QPT = _HQ // _NTILES
_G = 64
_NGR = _QPT // _G
_POOL = 2048


def _tc_block(q_ref, kT_ref, sm_ref, t0_ref):
    q = q_ref[...]
    qx = q[:, 0:1]
    qy = q[:, 1:2]
    qz = q[:, 2:3]
    q2 = qx * qx + qy * qy + qz * qz
    qb = q.astype(jnp.bfloat16)

    sms = []
    for c in range(_NPAD // _CK):
        kTc = kT_ref[:, c * _CK:(c + 1) * _CK]
        kx = kTc[0:1, :]
        ky = kTc[1:2, :]
        kz = kTc[2:3, :]
        k2 = kx * kx + ky * ky + kz * kz
        dot = lax.dot_general(qb, kTc.astype(jnp.bfloat16),
                              (((1,), (0,)), ((), ())),
                              preferred_element_type=jnp.float32)
        sq = jnp.maximum((q2 + k2) - 2.0 * dot, 0.0)
        sms.append(jnp.min(sq.reshape(_BQ, _CK // _SEG, _SEG), axis=2))
    sm = jnp.concatenate(sms, axis=1)
    sm_ref[...] = sm

    iota_s = jax.lax.broadcasted_iota(jnp.int32, (_BQ, _NS), 1).astype(jnp.float32)
    mv = jnp.min(sm, axis=1, keepdims=True)
    for _ in range(_K - 1):
        first = jnp.min(jnp.where(sm == mv, iota_s, _BIG), axis=1, keepdims=True)
        sm = jnp.where(iota_s == first, _INF, sm)
        mv = jnp.min(sm, axis=1, keepdims=True)
    t0_ref[...] = jnp.broadcast_to(mv, (_BQ, 8))


def _sc_select(qoff, sm_hbm, t0_hbm, kx_hbm, ky_hbm, kz_hbm, k2_hbm, out_hbm,
               kxv, kyv, kzv, k2v, smb, t0b, outb, segbuf, poold, pooli):
    wid = lax.axis_index("s") * 2 + lax.axis_index("c")
    base = wid * _QPT
    pltpu.sync_copy(kx_hbm, kxv)
    pltpu.sync_copy(ky_hbm, kyv)
    pltpu.sync_copy(kz_hbm, kzv)
    pltpu.sync_copy(k2_hbm, k2v)

    nq = jnp.maximum(jnp.minimum(_N - (qoff + base), _QPT), 0)
    iota = jax.lax.iota(jnp.int32, 16)

    def group_body(g, _):
        q0 = base + g * _G
        pltpu.sync_copy(sm_hbm.at[pl.ds(q0 * _NS, _G * _NS)], smb)
        pltpu.sync_copy(t0_hbm.at[pl.ds(q0 * 8, _G * 8)], t0b)
        nql = jnp.minimum(_G, nq - g * _G)

        def query_body(ql, _):
            t0v = plsc.load_gather(t0b, [jnp.full((16,), ql * 8, jnp.int32)])
            t0m = t0v * 1.00001 + 1e-4
            qg = jnp.full((16,), qoff + base + g * _G + ql, jnp.int32)
            qxv = plsc.load_gather(kxv, [qg])
            qyv = plsc.load_gather(kyv, [qg])
            qzv = plsc.load_gather(kzv, [qg])
            q2v = plsc.load_gather(k2v, [qg])

            scnt = jnp.int32(0)
            for c in range(_NS // 16):
                m = plsc.load_gather(
                    smb, [jnp.full((16,), ql * _NS + c * 16, jnp.int32) + iota])
                msk = m <= t0m
                cs = plsc.cumsum(msk.astype(jnp.int32))
                plsc.store_scatter(segbuf, [scnt + cs - 1], iota + c * 16,
                                   mask=msk)
                scnt = scnt + jnp.max(cs)

            def seg_body(i, pcnt):
                sv = plsc.load_gather(segbuf, [jnp.full((16,), i, jnp.int32)])
                for sub in range(_SEG // 16):
                    kidx = sv * _SEG + sub * 16 + iota
                    kxg = plsc.load_gather(kxv, [kidx])
                    kyg = plsc.load_gather(kyv, [kidx])
                    kzg = plsc.load_gather(kzv, [kidx])
                    k2g = plsc.load_gather(k2v, [kidx])
                    dot = (qxv * kxg + qyv * kyg) + qzv * kzg
                    sq = jnp.maximum((q2v + k2g) - 2.0 * dot, 0.0)
                    msk = sq <= t0m
                    cs = plsc.cumsum(msk.astype(jnp.int32))
                    pos = jnp.minimum(pcnt + cs - 1, _POOL + 15)
                    plsc.store_scatter(poold, [pos], sq, mask=msk)
                    plsc.store_scatter(pooli, [pos], kidx, mask=msk)
                    pcnt = pcnt + jnp.max(cs)
                return pcnt

            pcnt = lax.fori_loop(0, scnt, seg_body, jnp.int32(0))
            pc = jnp.minimum(pcnt, _POOL)
            plsc.store_scatter(poold, [jnp.full((16,), pc, jnp.int32) + iota],
                               jnp.full((16,), _INF, jnp.float32))

            bd, bi = plsc.sort_key_val(poold[pl.ds(0, 16)], pooli[pl.ds(0, 16)])

            def mrg(c, carry):
                bd, bi = carry
                cidx = jnp.full((16,), c * 16, jnp.int32) + iota
                cd, ci = plsc.sort_key_val(plsc.load_gather(poold, [cidx]),
                                           plsc.load_gather(pooli, [cidx]),
                                           descending=True)
                sel = cd < bd
                md = jnp.where(sel, cd, bd)
                mi = jnp.where(sel, ci, bi)
                return tuple(plsc.sort_key_val(md, mi))

            bd, bi = lax.fori_loop(1, (pc + 15) // 16, mrg, (bd, bi))
            plsc.store_scatter(outb, [jnp.full((16,), ql * _K, jnp.int32) + iota],
                               bi.astype(jnp.float32))
            return 0

        lax.fori_loop(0, nql, query_body, 0)
        pltpu.sync_copy(outb, out_hbm.at[pl.ds(q0 * _K, _G * _K)])
        return 0

    lax.fori_loop(0, _NGR, group_body, 0)


@jax.jit
def kernel(barycenters):
    n = barycenters.shape[0]
    pad = jnp.full((_NPAD - n, 3), _PADVAL, dtype=jnp.float32)
    bpad = jnp.concatenate([barycenters, pad], axis=0)
    kT = bpad.T

    kx = bpad[:, 0]
    ky = bpad[:, 1]
    kz = bpad[:, 2]
    k2 = kx * kx + ky * ky + kz * kz

    def bf16_round(x):
        b = lax.bitcast_convert_type(x, jnp.int32)
        r = (b + 0x7FFF + ((b >> 16) & 1)) & jnp.int32(-65536)
        return lax.bitcast_convert_type(r, jnp.float32)

    kxt = bf16_round(kx)
    kyt = bf16_round(ky)
    kzt = bf16_round(kz)

    nhb = _HQ // _BQ
    outs = []
    for h in range(_NHALF):
        sm, t0 = pl.pallas_call(
            _tc_block,
            grid=(nhb,),
            in_specs=[
                pl.BlockSpec((_BQ, 3), lambda i, h=h: (i + h * nhb, 0)),
                pl.BlockSpec((3, _NPAD), lambda i: (0, 0)),
            ],
            out_specs=[
                pl.BlockSpec((_BQ, _NS), lambda i: (i, 0)),
                pl.BlockSpec((_BQ, 8), lambda i: (i, 0)),
            ],
            out_shape=[
                jax.ShapeDtypeStruct((_HQ, _NS), jnp.float32),
                jax.ShapeDtypeStruct((_HQ, 8), jnp.float32),
            ],
        )(bpad, kT)

        sc = pl.kernel(
            functools.partial(_sc_select, h * _HQ),
            out_type=jax.ShapeDtypeStruct((_HQ * _K,), jnp.float32),
            mesh=plsc.VectorSubcoreMesh(core_axis_name="c",
                                        subcore_axis_name="s",
                                        num_cores=2, num_subcores=16),
            compiler_params=pltpu.CompilerParams(needs_layout_passes=False),
            scratch_types=[
                pltpu.VMEM((_NPAD,), jnp.float32),
                pltpu.VMEM((_NPAD,), jnp.float32),
                pltpu.VMEM((_NPAD,), jnp.float32),
                pltpu.VMEM((_NPAD,), jnp.float32),
                pltpu.VMEM((_G * _NS,), jnp.float32),
                pltpu.VMEM((_G * 8,), jnp.float32),
                pltpu.VMEM((_G * _K,), jnp.float32),
                pltpu.VMEM((_NS + 16,), jnp.int32),
                pltpu.VMEM((_POOL + 16,), jnp.float32),
                pltpu.VMEM((_POOL + 16,), jnp.int32),
            ],
        )
        outs.append(sc(sm.reshape(-1), t0.reshape(-1), kxt, kyt, kzt, k2))
    return jnp.concatenate(outs).reshape(_NPAD, _K)[:n]


# speedup vs baseline: 1.7361x; 1.0236x over previous
import functools

import jax
import jax.numpy as jnp
from jax import lax
from jax.experimental import pallas as pl
from jax.experimental.pallas import tpu as pltpu
from jax.experimental.pallas import tpu_sc as plsc

_K = 16
_N = 20000
_NPAD = 20480
_BQ = 128
_CK = 2048
_SEG = 128
_NS = _NPAD // _SEG
_PADVAL = 1.0e18
_INF = 3.0e38
_BIG = 1.0e9

_NTILES = 32
_NHALF = 2
_HQ = _NPAD // _NHALF
_QPT = _HQ // _NTILES
_G = 64
_NGR = _QPT // _G
_POOL = 2048


def _tc_block(q_ref, kT_ref, sm_ref, t0_ref):
    q = q_ref[...]
    qx = q[:, 0:1]
    qy = q[:, 1:2]
    qz = q[:, 2:3]
    q2 = qx * qx + qy * qy + qz * qz
    qb = q.astype(jnp.bfloat16)

    sms = []
    for c in range(_NPAD // _CK):
        kTc = kT_ref[:, c * _CK:(c + 1) * _CK]
        kx = kTc[0:1, :]
        ky = kTc[1:2, :]
        kz = kTc[2:3, :]
        k2 = kx * kx + ky * ky + kz * kz
        dot = lax.dot_general(qb, kTc.astype(jnp.bfloat16),
                              (((1,), (0,)), ((), ())),
                              preferred_element_type=jnp.float32)
        sq = jnp.maximum((q2 + k2) - 2.0 * dot, 0.0)
        sms.append(jnp.min(sq.reshape(_BQ, _CK // _SEG, _SEG), axis=2))
    sm = jnp.concatenate(sms, axis=1)
    sm_ref[...] = sm

    iota_s = jax.lax.broadcasted_iota(jnp.int32, (_BQ, _NS), 1).astype(jnp.float32)
    mv = jnp.min(sm, axis=1, keepdims=True)
    for _ in range(_K - 1):
        first = jnp.min(jnp.where(sm == mv, iota_s, _BIG), axis=1, keepdims=True)
        sm = jnp.where(iota_s == first, _INF, sm)
        mv = jnp.min(sm, axis=1, keepdims=True)
    t0_ref[...] = jnp.broadcast_to(mv, (_BQ, 8))


def _sc_select(qoff, sm_hbm, t0_hbm, kx_hbm, ky_hbm, kz_hbm, k2_hbm, out_hbm,
               kxv, kyv, kzv, k2v, smb, t0b, outb, segbuf, poold, pooli):
    wid = lax.axis_index("s") * 2 + lax.axis_index("c")
    base = wid * _QPT
    pltpu.sync_copy(kx_hbm, kxv)
    pltpu.sync_copy(ky_hbm, kyv)
    pltpu.sync_copy(kz_hbm, kzv)
    pltpu.sync_copy(k2_hbm, k2v)

    nq = jnp.maximum(jnp.minimum(_N - (qoff + base), _QPT), 0)
    iota = jax.lax.iota(jnp.int32, 16)

    def group_body(g, _):
        q0 = base + g * _G
        pltpu.sync_copy(sm_hbm.at[pl.ds(q0 * _NS, _G * _NS)], smb)
        pltpu.sync_copy(t0_hbm.at[pl.ds(q0 * 8, _G * 8)], t0b)
        nql = jnp.minimum(_G, nq - g * _G)

        def query_body(ql, _):
            t0v = plsc.load_gather(t0b, [jnp.full((16,), ql * 8, jnp.int32)])
            t0m = t0v * 1.00001 + 1e-4
            qg = jnp.full((16,), qoff + base + g * _G + ql, jnp.int32)
            qxv = plsc.load_gather(kxv, [qg])
            qyv = plsc.load_gather(kyv, [qg])
            qzv = plsc.load_gather(kzv, [qg])
            q2v = plsc.load_gather(k2v, [qg])

            scnt_v = jnp.zeros((16,), jnp.int32)
            for c in range(_NS // 16):
                m = plsc.load_gather(
                    smb, [jnp.full((16,), ql * _NS + c * 16, jnp.int32) + iota])
                msk = m <= t0m
                cs = plsc.cumsum(msk.astype(jnp.int32))
                plsc.store_scatter(segbuf, [scnt_v + cs - 1], iota + c * 16,
                                   mask=msk)
                scnt_v = scnt_v + plsc.all_reduce_population_count(msk)
            scnt = jnp.max(scnt_v)

            def seg_body(i, pcnt_v):
                sv = plsc.load_gather(segbuf, [jnp.full((16,), i, jnp.int32)])
                for sub in range(_SEG // 16):
                    kidx = sv * _SEG + sub * 16 + iota
                    kxg = plsc.load_gather(kxv, [kidx])
                    kyg = plsc.load_gather(kyv, [kidx])
                    kzg = plsc.load_gather(kzv, [kidx])
                    k2g = plsc.load_gather(k2v, [kidx])
                    dot = (qxv * kxg + qyv * kyg) + qzv * kzg
                    sq = jnp.maximum((q2v + k2g) - 2.0 * dot, 0.0)
                    msk = sq <= t0m
                    cs = plsc.cumsum(msk.astype(jnp.int32))
                    pos = jnp.minimum(pcnt_v + cs - 1, _POOL + 15)
                    plsc.store_scatter(poold, [pos], sq, mask=msk)
                    plsc.store_scatter(pooli, [pos], kidx, mask=msk)
                    pcnt_v = pcnt_v + plsc.all_reduce_population_count(msk)
                return pcnt_v

            pcnt_v = lax.fori_loop(0, scnt, seg_body, jnp.zeros((16,), jnp.int32))
            pc = jnp.minimum(jnp.max(pcnt_v), _POOL)
            plsc.store_scatter(poold, [jnp.full((16,), pc, jnp.int32) + iota],
                               jnp.full((16,), _INF, jnp.float32))

            bd, bi = plsc.sort_key_val(poold[pl.ds(0, 16)], pooli[pl.ds(0, 16)])

            def mrg(c, carry):
                bd, bi = carry
                cidx = jnp.full((16,), c * 16, jnp.int32) + iota
                cd, ci = plsc.sort_key_val(plsc.load_gather(poold, [cidx]),
                                           plsc.load_gather(pooli, [cidx]),
                                           descending=True)
                sel = cd < bd
                md = jnp.where(sel, cd, bd)
                mi = jnp.where(sel, ci, bi)
                return tuple(plsc.sort_key_val(md, mi))

            bd, bi = lax.fori_loop(1, (pc + 15) // 16, mrg, (bd, bi))
            plsc.store_scatter(outb, [jnp.full((16,), ql * _K, jnp.int32) + iota],
                               bi.astype(jnp.float32))
            return 0

        lax.fori_loop(0, nql, query_body, 0)
        pltpu.sync_copy(outb, out_hbm.at[pl.ds(q0 * _K, _G * _K)])
        return 0

    lax.fori_loop(0, _NGR, group_body, 0)


@jax.jit
def kernel(barycenters):
    n = barycenters.shape[0]
    pad = jnp.full((_NPAD - n, 3), _PADVAL, dtype=jnp.float32)
    bpad = jnp.concatenate([barycenters, pad], axis=0)
    kT = bpad.T

    kx = bpad[:, 0]
    ky = bpad[:, 1]
    kz = bpad[:, 2]
    k2 = kx * kx + ky * ky + kz * kz

    def bf16_round(x):
        b = lax.bitcast_convert_type(x, jnp.int32)
        r = (b + 0x7FFF + ((b >> 16) & 1)) & jnp.int32(-65536)
        return lax.bitcast_convert_type(r, jnp.float32)

    kxt = bf16_round(kx)
    kyt = bf16_round(ky)
    kzt = bf16_round(kz)

    nhb = _HQ // _BQ
    outs = []
    for h in range(_NHALF):
        sm, t0 = pl.pallas_call(
            _tc_block,
            grid=(nhb,),
            in_specs=[
                pl.BlockSpec((_BQ, 3), lambda i, h=h: (i + h * nhb, 0)),
                pl.BlockSpec((3, _NPAD), lambda i: (0, 0)),
            ],
            out_specs=[
                pl.BlockSpec((_BQ, _NS), lambda i: (i, 0)),
                pl.BlockSpec((_BQ, 8), lambda i: (i, 0)),
            ],
            out_shape=[
                jax.ShapeDtypeStruct((_HQ, _NS), jnp.float32),
                jax.ShapeDtypeStruct((_HQ, 8), jnp.float32),
            ],
        )(bpad, kT)

        sc = pl.kernel(
            functools.partial(_sc_select, h * _HQ),
            out_type=jax.ShapeDtypeStruct((_HQ * _K,), jnp.float32),
            mesh=plsc.VectorSubcoreMesh(core_axis_name="c",
                                        subcore_axis_name="s",
                                        num_cores=2, num_subcores=16),
            compiler_params=pltpu.CompilerParams(needs_layout_passes=False),
            scratch_types=[
                pltpu.VMEM((_NPAD,), jnp.float32),
                pltpu.VMEM((_NPAD,), jnp.float32),
                pltpu.VMEM((_NPAD,), jnp.float32),
                pltpu.VMEM((_NPAD,), jnp.float32),
                pltpu.VMEM((_G * _NS,), jnp.float32),
                pltpu.VMEM((_G * 8,), jnp.float32),
                pltpu.VMEM((_G * _K,), jnp.float32),
                pltpu.VMEM((_NS + 16,), jnp.int32),
                pltpu.VMEM((_POOL + 16,), jnp.float32),
                pltpu.VMEM((_POOL + 16,), jnp.int32),
            ],
        )
        outs.append(sc(sm.reshape(-1), t0.reshape(-1), kxt, kyt, kzt, k2))
    return jnp.concatenate(outs).reshape(_NPAD, _K)[:n]


# speedup vs baseline: 1.9623x; 1.1303x over previous
import functools

import jax
import jax.numpy as jnp
from jax import lax
from jax.experimental import pallas as pl
from jax.experimental.pallas import tpu as pltpu
from jax.experimental.pallas import tpu_sc as plsc

_K = 16
_N = 20000
_NPAD = 20480
_BQ = 128
_CK = 2048
_SEG = 128
_NS = _NPAD // _SEG
_NSF = _NPAD // 16
_PADVAL = 1.0e18
_INF = 3.0e38
_BIG = 1.0e9

_NTILES = 32
_NHALF = 2
_HQ = _NPAD // _NHALF
_QPT = _HQ // _NTILES
_G = 8
_NGR = _QPT // _G
_POOL = 2048


def _tc_block(q_ref, kT_ref, sm16_ref, t0_ref):
    q = q_ref[...]
    qx = q[:, 0:1]
    qy = q[:, 1:2]
    qz = q[:, 2:3]
    q2 = qx * qx + qy * qy + qz * qz
    qb = q.astype(jnp.bfloat16)

    sms = []
    sm16s = []
    for c in range(_NPAD // _CK):
        kTc = kT_ref[:, c * _CK:(c + 1) * _CK]
        kx = kTc[0:1, :]
        ky = kTc[1:2, :]
        kz = kTc[2:3, :]
        k2 = kx * kx + ky * ky + kz * kz
        dot = lax.dot_general(qb, kTc.astype(jnp.bfloat16),
                              (((1,), (0,)), ((), ())),
                              preferred_element_type=jnp.float32)
        sq = jnp.maximum((q2 + k2) - 2.0 * dot, 0.0)
        sq3 = sq.reshape(_BQ, _CK // _SEG, _SEG)
        sms.append(jnp.min(sq3, axis=2))
        sm16s.append(jnp.min(sq3, axis=1))
    sm = jnp.concatenate(sms, axis=1)
    sm16_ref[...] = jnp.concatenate(sm16s, axis=1)

    iota_s = jax.lax.broadcasted_iota(jnp.int32, (_BQ, _NS), 1).astype(jnp.float32)
    mv = jnp.min(sm, axis=1, keepdims=True)
    for _ in range(_K - 1):
        first = jnp.min(jnp.where(sm == mv, iota_s, _BIG), axis=1, keepdims=True)
        sm = jnp.where(iota_s == first, _INF, sm)
        mv = jnp.min(sm, axis=1, keepdims=True)
    t0_ref[...] = jnp.broadcast_to(mv, (_BQ, 8))


def _sc_select(qoff, sm_hbm, t0_hbm, kx_hbm, ky_hbm, kz_hbm, k2_hbm, out_hbm,
               kxv, kyv, kzv, k2v, smb, t0b, outb, segbuf, poold, pooli):
    wid = lax.axis_index("s") * 2 + lax.axis_index("c")
    base = wid * _QPT
    pltpu.sync_copy(kx_hbm, kxv)
    pltpu.sync_copy(ky_hbm, kyv)
    pltpu.sync_copy(kz_hbm, kzv)
    pltpu.sync_copy(k2_hbm, k2v)

    nq = jnp.maximum(jnp.minimum(_N - (qoff + base), _QPT), 0)
    iota = jax.lax.iota(jnp.int32, 16)

    def group_body(g, _):
        q0 = base + g * _G
        pltpu.sync_copy(sm_hbm.at[pl.ds(q0 * _NSF, _G * _NSF)], smb)
        pltpu.sync_copy(t0_hbm.at[pl.ds(q0 * 8, _G * 8)], t0b)
        nql = jnp.minimum(_G, nq - g * _G)

        def query_body(ql, _):
            t0v = plsc.load_gather(t0b, [jnp.full((16,), ql * 8, jnp.int32)])
            t0m = t0v * 1.00001 + 1e-4
            qg = jnp.full((16,), qoff + base + g * _G + ql, jnp.int32)
            qxv = plsc.load_gather(kxv, [qg])
            qyv = plsc.load_gather(kyv, [qg])
            qzv = plsc.load_gather(kzv, [qg])
            q2v = plsc.load_gather(k2v, [qg])

            scnt_v = jnp.zeros((16,), jnp.int32)
            for c in range(_NSF // 16):
                m = plsc.load_gather(
                    smb, [jnp.full((16,), ql * _NSF + c * 16, jnp.int32) + iota])
                msk = m <= t0m
                cs = plsc.cumsum(msk.astype(jnp.int32))
                plsc.store_scatter(segbuf, [scnt_v + cs - 1], iota + c * 16,
                                   mask=msk)
                scnt_v = scnt_v + plsc.all_reduce_population_count(msk)
            scnt = jnp.max(scnt_v)

            def seg_body(i, pcnt_v):
                fv = plsc.load_gather(segbuf, [jnp.full((16,), i, jnp.int32)])
                kidx = ((fv >> 7) * 2048 + (fv & 127)) + (iota << 7)
                kxg = plsc.load_gather(kxv, [kidx])
                kyg = plsc.load_gather(kyv, [kidx])
                kzg = plsc.load_gather(kzv, [kidx])
                k2g = plsc.load_gather(k2v, [kidx])
                dot = (qxv * kxg + qyv * kyg) + qzv * kzg
                sq = jnp.maximum((q2v + k2g) - 2.0 * dot, 0.0)
                msk = sq <= t0m
                cs = plsc.cumsum(msk.astype(jnp.int32))
                pos = jnp.minimum(pcnt_v + cs - 1, _POOL + 15)
                plsc.store_scatter(poold, [pos], sq, mask=msk)
                plsc.store_scatter(pooli, [pos], kidx, mask=msk)
                return pcnt_v + plsc.all_reduce_population_count(msk)

            pcnt_v = lax.fori_loop(0, scnt, seg_body, jnp.zeros((16,), jnp.int32))
            pc = jnp.minimum(jnp.max(pcnt_v), _POOL)
            plsc.store_scatter(poold, [jnp.full((16,), pc, jnp.int32) + iota],
                               jnp.full((16,), _INF, jnp.float32))

            bd, bi = plsc.sort_key_val(poold[pl.ds(0, 16)], pooli[pl.ds(0, 16)])

            def mrg(c, carry):
                bd, bi = carry
                cidx = jnp.full((16,), c * 16, jnp.int32) + iota
                cd, ci = plsc.sort_key_val(plsc.load_gather(poold, [cidx]),
                                           plsc.load_gather(pooli, [cidx]),
                                           descending=True)
                sel = cd < bd
                md = jnp.where(sel, cd, bd)
                mi = jnp.where(sel, ci, bi)
                return tuple(plsc.sort_key_val(md, mi))

            bd, bi = lax.fori_loop(1, (pc + 15) // 16, mrg, (bd, bi))
            plsc.store_scatter(outb, [jnp.full((16,), ql * _K, jnp.int32) + iota],
                               bi.astype(jnp.float32))
            return 0

        lax.fori_loop(0, nql, query_body, 0)
        pltpu.sync_copy(outb, out_hbm.at[pl.ds(q0 * _K, _G * _K)])
        return 0

    lax.fori_loop(0, _NGR, group_body, 0)


@jax.jit
def kernel(barycenters):
    n = barycenters.shape[0]
    pad = jnp.full((_NPAD - n, 3), _PADVAL, dtype=jnp.float32)
    bpad = jnp.concatenate([barycenters, pad], axis=0)
    kT = bpad.T

    kx = bpad[:, 0]
    ky = bpad[:, 1]
    kz = bpad[:, 2]
    k2 = kx * kx + ky * ky + kz * kz

    def bf16_round(x):
        b = lax.bitcast_convert_type(x, jnp.int32)
        r = (b + 0x7FFF + ((b >> 16) & 1)) & jnp.int32(-65536)
        return lax.bitcast_convert_type(r, jnp.float32)

    kxt = bf16_round(kx)
    kyt = bf16_round(ky)
    kzt = bf16_round(kz)

    nhb = _HQ // _BQ
    outs = []
    for h in range(_NHALF):
        sm, t0 = pl.pallas_call(
            _tc_block,
            grid=(nhb,),
            in_specs=[
                pl.BlockSpec((_BQ, 3), lambda i, h=h: (i + h * nhb, 0)),
                pl.BlockSpec((3, _NPAD), lambda i: (0, 0)),
            ],
            out_specs=[
                pl.BlockSpec((_BQ, _NSF), lambda i: (i, 0)),
                pl.BlockSpec((_BQ, 8), lambda i: (i, 0)),
            ],
            out_shape=[
                jax.ShapeDtypeStruct((_HQ, _NSF), jnp.float32),
                jax.ShapeDtypeStruct((_HQ, 8), jnp.float32),
            ],
        )(bpad, kT)

        sc = pl.kernel(
            functools.partial(_sc_select, h * _HQ),
            out_type=jax.ShapeDtypeStruct((_HQ * _K,), jnp.float32),
            mesh=plsc.VectorSubcoreMesh(core_axis_name="c",
                                        subcore_axis_name="s",
                                        num_cores=2, num_subcores=16),
            compiler_params=pltpu.CompilerParams(needs_layout_passes=False),
            scratch_types=[
                pltpu.VMEM((_NPAD,), jnp.float32),
                pltpu.VMEM((_NPAD,), jnp.float32),
                pltpu.VMEM((_NPAD,), jnp.float32),
                pltpu.VMEM((_NPAD,), jnp.float32),
                pltpu.VMEM((_G * _NSF,), jnp.float32),
                pltpu.VMEM((_G * 8,), jnp.float32),
                pltpu.VMEM((_G * _K,), jnp.float32),
                pltpu.VMEM((_NSF + 16,), jnp.int32),
                pltpu.VMEM((_POOL + 16,), jnp.float32),
                pltpu.VMEM((_POOL + 16,), jnp.int32),
            ],
        )
        outs.append(sc(sm.reshape(-1), t0.reshape(-1), kxt, kyt, kzt, k2))
    return jnp.concatenate(outs).reshape(_NPAD, _K)[:n]


# speedup vs baseline: 2.3123x; 1.1784x over previous
import functools

import jax
import jax.numpy as jnp
from jax import lax
from jax.experimental import pallas as pl
from jax.experimental.pallas import tpu as pltpu
from jax.experimental.pallas import tpu_sc as plsc

_K = 16
_N = 20000
_NPAD = 20480
_BQ = 128
_CK = 2048
_SEG = 128
_NS = _NPAD // _SEG
_NSF = _NPAD // 16
_PADVAL = 1.0e18
_INF = 3.0e38
_BIG = 1.0e9

_NTILES = 32
_NHALF = 4
_HQ = _NPAD // _NHALF
_QPT = _HQ // _NTILES
_G = 8
_NGR = _QPT // _G
_POOL = 2048


def _tc_block(q_ref, kT_ref, sm16_ref, t0_ref):
    q = q_ref[...]
    qx = q[:, 0:1]
    qy = q[:, 1:2]
    qz = q[:, 2:3]
    q2 = qx * qx + qy * qy + qz * qz
    qb = q.astype(jnp.bfloat16)

    sms = []
    sm16s = []
    for c in range(_NPAD // _CK):
        kTc = kT_ref[:, c * _CK:(c + 1) * _CK]
        kx = kTc[0:1, :]
        ky = kTc[1:2, :]
        kz = kTc[2:3, :]
        k2 = kx * kx + ky * ky + kz * kz
        dot = lax.dot_general(qb, kTc.astype(jnp.bfloat16),
                              (((1,), (0,)), ((), ())),
                              preferred_element_type=jnp.float32)
        sq = jnp.maximum((q2 + k2) - 2.0 * dot, 0.0)
        sq3 = sq.reshape(_BQ, _CK // _SEG, _SEG)
        sms.append(jnp.min(sq3, axis=2))
        sm16s.append(jnp.min(sq3, axis=1))
    sm = jnp.concatenate(sms, axis=1)
    sm16_ref[...] = jnp.concatenate(sm16s, axis=1)

    iota_s = jax.lax.broadcasted_iota(jnp.int32, (_BQ, _NS), 1).astype(jnp.float32)
    mv = jnp.min(sm, axis=1, keepdims=True)
    for _ in range(_K - 1):
        first = jnp.min(jnp.where(sm == mv, iota_s, _BIG), axis=1, keepdims=True)
        sm = jnp.where(iota_s == first, _INF, sm)
        mv = jnp.min(sm, axis=1, keepdims=True)
    t0_ref[...] = jnp.broadcast_to(mv, (_BQ, 8))


def _sc_select(qoff, sm_hbm, t0_hbm, kx_hbm, ky_hbm, kz_hbm, k2_hbm, out_hbm,
               kxv, kyv, kzv, k2v, smb, t0b, outb, segbuf, poold, pooli):
    wid = lax.axis_index("s") * 2 + lax.axis_index("c")
    base = wid * _QPT
    pltpu.sync_copy(kx_hbm, kxv)
    pltpu.sync_copy(ky_hbm, kyv)
    pltpu.sync_copy(kz_hbm, kzv)
    pltpu.sync_copy(k2_hbm, k2v)

    nq = jnp.maximum(jnp.minimum(_N - (qoff + base), _QPT), 0)
    iota = jax.lax.iota(jnp.int32, 16)

    def group_body(g, _):
        q0 = base + g * _G
        pltpu.sync_copy(sm_hbm.at[pl.ds(q0 * _NSF, _G * _NSF)], smb)
        pltpu.sync_copy(t0_hbm.at[pl.ds(q0 * 8, _G * 8)], t0b)
        nql = jnp.minimum(_G, nq - g * _G)

        def query_body(ql, _):
            t0v = plsc.load_gather(t0b, [jnp.full((16,), ql * 8, jnp.int32)])
            t0m = t0v * 1.00001 + 1e-4
            qg = jnp.full((16,), qoff + base + g * _G + ql, jnp.int32)
            qxv = plsc.load_gather(kxv, [qg])
            qyv = plsc.load_gather(kyv, [qg])
            qzv = plsc.load_gather(kzv, [qg])
            q2v = plsc.load_gather(k2v, [qg])

            scnt_v = jnp.zeros((16,), jnp.int32)
            for c in range(_NSF // 16):
                m = plsc.load_gather(
                    smb, [jnp.full((16,), ql * _NSF + c * 16, jnp.int32) + iota])
                msk = m <= t0m
                cs = plsc.cumsum(msk.astype(jnp.int32))
                plsc.store_scatter(segbuf, [scnt_v + cs - 1], iota + c * 16,
                                   mask=msk)
                scnt_v = scnt_v + plsc.all_reduce_population_count(msk)
            scnt = jnp.max(scnt_v)

            def seg_body(i, pcnt_v):
                fv = plsc.load_gather(segbuf, [jnp.full((16,), i, jnp.int32)])
                kidx = ((fv >> 7) * 2048 + (fv & 127)) + (iota << 7)
                kxg = plsc.load_gather(kxv, [kidx])
                kyg = plsc.load_gather(kyv, [kidx])
                kzg = plsc.load_gather(kzv, [kidx])
                k2g = plsc.load_gather(k2v, [kidx])
                dot = (qxv * kxg + qyv * kyg) + qzv * kzg
                sq = jnp.maximum((q2v + k2g) - 2.0 * dot, 0.0)
                msk = sq <= t0m
                cs = plsc.cumsum(msk.astype(jnp.int32))
                pos = jnp.minimum(pcnt_v + cs - 1, _POOL + 15)
                plsc.store_scatter(poold, [pos], sq, mask=msk)
                plsc.store_scatter(pooli, [pos], kidx, mask=msk)
                return pcnt_v + plsc.all_reduce_population_count(msk)

            pcnt_v = lax.fori_loop(0, scnt, seg_body, jnp.zeros((16,), jnp.int32))
            pc = jnp.minimum(jnp.max(pcnt_v), _POOL)
            plsc.store_scatter(poold, [jnp.full((16,), pc, jnp.int32) + iota],
                               jnp.full((16,), _INF, jnp.float32))

            bd, bi = plsc.sort_key_val(poold[pl.ds(0, 16)], pooli[pl.ds(0, 16)])

            def mrg(c, carry):
                bd, bi = carry
                cidx = jnp.full((16,), c * 16, jnp.int32) + iota
                cd, ci = plsc.sort_key_val(plsc.load_gather(poold, [cidx]),
                                           plsc.load_gather(pooli, [cidx]),
                                           descending=True)
                sel = cd < bd
                md = jnp.where(sel, cd, bd)
                mi = jnp.where(sel, ci, bi)
                return tuple(plsc.sort_key_val(md, mi))

            bd, bi = lax.fori_loop(1, (pc + 15) // 16, mrg, (bd, bi))
            plsc.store_scatter(outb, [jnp.full((16,), ql * _K, jnp.int32) + iota],
                               bi.astype(jnp.float32))
            return 0

        lax.fori_loop(0, nql, query_body, 0)
        pltpu.sync_copy(outb, out_hbm.at[pl.ds(q0 * _K, _G * _K)])
        return 0

    lax.fori_loop(0, _NGR, group_body, 0)


@jax.jit
def kernel(barycenters):
    n = barycenters.shape[0]
    pad = jnp.full((_NPAD - n, 3), _PADVAL, dtype=jnp.float32)
    bpad = jnp.concatenate([barycenters, pad], axis=0)
    kT = bpad.T

    kx = bpad[:, 0]
    ky = bpad[:, 1]
    kz = bpad[:, 2]
    k2 = kx * kx + ky * ky + kz * kz

    def bf16_round(x):
        b = lax.bitcast_convert_type(x, jnp.int32)
        r = (b + 0x7FFF + ((b >> 16) & 1)) & jnp.int32(-65536)
        return lax.bitcast_convert_type(r, jnp.float32)

    kxt = bf16_round(kx)
    kyt = bf16_round(ky)
    kzt = bf16_round(kz)

    nhb = _HQ // _BQ
    outs = []
    for h in range(_NHALF):
        sm, t0 = pl.pallas_call(
            _tc_block,
            grid=(nhb,),
            in_specs=[
                pl.BlockSpec((_BQ, 3), lambda i, h=h: (i + h * nhb, 0)),
                pl.BlockSpec((3, _NPAD), lambda i: (0, 0)),
            ],
            out_specs=[
                pl.BlockSpec((_BQ, _NSF), lambda i: (i, 0)),
                pl.BlockSpec((_BQ, 8), lambda i: (i, 0)),
            ],
            out_shape=[
                jax.ShapeDtypeStruct((_HQ, _NSF), jnp.float32),
                jax.ShapeDtypeStruct((_HQ, 8), jnp.float32),
            ],
        )(bpad, kT)

        sc = pl.kernel(
            functools.partial(_sc_select, h * _HQ),
            out_type=jax.ShapeDtypeStruct((_HQ * _K,), jnp.float32),
            mesh=plsc.VectorSubcoreMesh(core_axis_name="c",
                                        subcore_axis_name="s",
                                        num_cores=2, num_subcores=16),
            compiler_params=pltpu.CompilerParams(needs_layout_passes=False),
            scratch_types=[
                pltpu.VMEM((_NPAD,), jnp.float32),
                pltpu.VMEM((_NPAD,), jnp.float32),
                pltpu.VMEM((_NPAD,), jnp.float32),
                pltpu.VMEM((_NPAD,), jnp.float32),
                pltpu.VMEM((_G * _NSF,), jnp.float32),
                pltpu.VMEM((_G * 8,), jnp.float32),
                pltpu.VMEM((_G * _K,), jnp.float32),
                pltpu.VMEM((_NSF + 16,), jnp.int32),
                pltpu.VMEM((_POOL + 16,), jnp.float32),
                pltpu.VMEM((_POOL + 16,), jnp.int32),
            ],
        )
        outs.append(sc(sm.reshape(-1), t0.reshape(-1), kxt, kyt, kzt, k2))
    return jnp.concatenate(outs).reshape(_NPAD, _K)[:n]


# speedup vs baseline: 2.4620x; 1.0648x over previous
import functools

import jax
import jax.numpy as jnp
from jax import lax
from jax.experimental import pallas as pl
from jax.experimental.pallas import tpu as pltpu
from jax.experimental.pallas import tpu_sc as plsc

_K = 16
_N = 20000
_NPAD = 20480
_BQ = 128
_CK = 2048
_SEG = 128
_NS = _NPAD // _SEG
_NSF = _NPAD // 16
_PADVAL = 1.0e18
_INF = 3.0e38
_BIG = 1.0e9

_NTILES = 32
_NHALF = 8
_HQ = _NPAD // _NHALF
_QPT = _HQ // _NTILES
_G = 8
_NGR = _QPT // _G
_POOL = 2048


def _tc_block(q_ref, kT_ref, sm16_ref, t0_ref):
    q = q_ref[...]
    qx = q[:, 0:1]
    qy = q[:, 1:2]
    qz = q[:, 2:3]
    q2 = qx * qx + qy * qy + qz * qz
    qb = q.astype(jnp.bfloat16)

    sms = []
    sm16s = []
    for c in range(_NPAD // _CK):
        kTc = kT_ref[:, c * _CK:(c + 1) * _CK]
        kx = kTc[0:1, :]
        ky = kTc[1:2, :]
        kz = kTc[2:3, :]
        k2 = kx * kx + ky * ky + kz * kz
        dot = lax.dot_general(qb, kTc.astype(jnp.bfloat16),
                              (((1,), (0,)), ((), ())),
                              preferred_element_type=jnp.float32)
        sq = jnp.maximum((q2 + k2) - 2.0 * dot, 0.0)
        sq3 = sq.reshape(_BQ, _CK // _SEG, _SEG)
        sms.append(jnp.min(sq3, axis=2))
        sm16s.append(jnp.min(sq3, axis=1))
    sm = jnp.concatenate(sms, axis=1)
    sm16_ref[...] = jnp.concatenate(sm16s, axis=1)

    iota_s = jax.lax.broadcasted_iota(jnp.int32, (_BQ, _NS), 1).astype(jnp.float32)
    mv = jnp.min(sm, axis=1, keepdims=True)
    for _ in range(_K - 1):
        first = jnp.min(jnp.where(sm == mv, iota_s, _BIG), axis=1, keepdims=True)
        sm = jnp.where(iota_s == first, _INF, sm)
        mv = jnp.min(sm, axis=1, keepdims=True)
    t0_ref[...] = jnp.broadcast_to(mv, (_BQ, 8))


def _sc_select(qoff, sm_hbm, t0_hbm, kx_hbm, ky_hbm, kz_hbm, k2_hbm, out_hbm,
               kxv, kyv, kzv, k2v, smb, t0b, outb, segbuf, poold, pooli):
    wid = lax.axis_index("s") * 2 + lax.axis_index("c")
    base = wid * _QPT
    pltpu.sync_copy(kx_hbm, kxv)
    pltpu.sync_copy(ky_hbm, kyv)
    pltpu.sync_copy(kz_hbm, kzv)
    pltpu.sync_copy(k2_hbm, k2v)

    nq = jnp.maximum(jnp.minimum(_N - (qoff + base), _QPT), 0)
    iota = jax.lax.iota(jnp.int32, 16)

    def group_body(g, _):
        q0 = base + g * _G
        pltpu.sync_copy(sm_hbm.at[pl.ds(q0 * _NSF, _G * _NSF)], smb)
        pltpu.sync_copy(t0_hbm.at[pl.ds(q0 * 8, _G * 8)], t0b)
        nql = jnp.minimum(_G, nq - g * _G)

        def query_body(ql, _):
            t0v = plsc.load_gather(t0b, [jnp.full((16,), ql * 8, jnp.int32)])
            t0m = t0v * 1.00001 + 1e-4
            qg = jnp.full((16,), qoff + base + g * _G + ql, jnp.int32)
            qxv = plsc.load_gather(kxv, [qg])
            qyv = plsc.load_gather(kyv, [qg])
            qzv = plsc.load_gather(kzv, [qg])
            q2v = plsc.load_gather(k2v, [qg])

            scnt_v = jnp.zeros((16,), jnp.int32)
            for c in range(_NSF // 16):
                m = plsc.load_gather(
                    smb, [jnp.full((16,), ql * _NSF + c * 16, jnp.int32) + iota])
                msk = m <= t0m
                cs = plsc.cumsum(msk.astype(jnp.int32))
                plsc.store_scatter(segbuf, [scnt_v + cs - 1], iota + c * 16,
                                   mask=msk)
                scnt_v = scnt_v + plsc.all_reduce_population_count(msk)
            scnt = jnp.max(scnt_v)

            def seg_body(i, pcnt_v):
                fv = plsc.load_gather(segbuf, [jnp.full((16,), i, jnp.int32)])
                kidx = ((fv >> 7) * 2048 + (fv & 127)) + (iota << 7)
                kxg = plsc.load_gather(kxv, [kidx])
                kyg = plsc.load_gather(kyv, [kidx])
                kzg = plsc.load_gather(kzv, [kidx])
                k2g = plsc.load_gather(k2v, [kidx])
                dot = (qxv * kxg + qyv * kyg) + qzv * kzg
                sq = jnp.maximum((q2v + k2g) - 2.0 * dot, 0.0)
                msk = sq <= t0m
                cs = plsc.cumsum(msk.astype(jnp.int32))
                pos = jnp.minimum(pcnt_v + cs - 1, _POOL + 15)
                plsc.store_scatter(poold, [pos], sq, mask=msk)
                plsc.store_scatter(pooli, [pos], kidx, mask=msk)
                return pcnt_v + plsc.all_reduce_population_count(msk)

            pcnt_v = lax.fori_loop(0, scnt, seg_body, jnp.zeros((16,), jnp.int32))
            pc = jnp.minimum(jnp.max(pcnt_v), _POOL)
            plsc.store_scatter(poold, [jnp.full((16,), pc, jnp.int32) + iota],
                               jnp.full((16,), _INF, jnp.float32))

            bd, bi = plsc.sort_key_val(poold[pl.ds(0, 16)], pooli[pl.ds(0, 16)])

            def mrg(c, carry):
                bd, bi = carry
                cidx = jnp.full((16,), c * 16, jnp.int32) + iota
                cd, ci = plsc.sort_key_val(plsc.load_gather(poold, [cidx]),
                                           plsc.load_gather(pooli, [cidx]),
                                           descending=True)
                sel = cd < bd
                md = jnp.where(sel, cd, bd)
                mi = jnp.where(sel, ci, bi)
                return tuple(plsc.sort_key_val(md, mi))

            bd, bi = lax.fori_loop(1, (pc + 15) // 16, mrg, (bd, bi))
            plsc.store_scatter(outb, [jnp.full((16,), ql * _K, jnp.int32) + iota],
                               bi.astype(jnp.float32))
            return 0

        lax.fori_loop(0, nql, query_body, 0)
        pltpu.sync_copy(outb, out_hbm.at[pl.ds(q0 * _K, _G * _K)])
        return 0

    lax.fori_loop(0, _NGR, group_body, 0)


@jax.jit
def kernel(barycenters):
    n = barycenters.shape[0]
    pad = jnp.full((_NPAD - n, 3), _PADVAL, dtype=jnp.float32)
    bpad = jnp.concatenate([barycenters, pad], axis=0)
    kT = bpad.T

    kx = bpad[:, 0]
    ky = bpad[:, 1]
    kz = bpad[:, 2]
    k2 = kx * kx + ky * ky + kz * kz

    def bf16_round(x):
        b = lax.bitcast_convert_type(x, jnp.int32)
        r = (b + 0x7FFF + ((b >> 16) & 1)) & jnp.int32(-65536)
        return lax.bitcast_convert_type(r, jnp.float32)

    kxt = bf16_round(kx)
    kyt = bf16_round(ky)
    kzt = bf16_round(kz)

    nhb = _HQ // _BQ
    outs = []
    for h in range(_NHALF):
        sm, t0 = pl.pallas_call(
            _tc_block,
            grid=(nhb,),
            in_specs=[
                pl.BlockSpec((_BQ, 3), lambda i, h=h: (i + h * nhb, 0)),
                pl.BlockSpec((3, _NPAD), lambda i: (0, 0)),
            ],
            out_specs=[
                pl.BlockSpec((_BQ, _NSF), lambda i: (i, 0)),
                pl.BlockSpec((_BQ, 8), lambda i: (i, 0)),
            ],
            out_shape=[
                jax.ShapeDtypeStruct((_HQ, _NSF), jnp.float32),
                jax.ShapeDtypeStruct((_HQ, 8), jnp.float32),
            ],
        )(bpad, kT)

        sc = pl.kernel(
            functools.partial(_sc_select, h * _HQ),
            out_type=jax.ShapeDtypeStruct((_HQ * _K,), jnp.float32),
            mesh=plsc.VectorSubcoreMesh(core_axis_name="c",
                                        subcore_axis_name="s",
                                        num_cores=2, num_subcores=16),
            compiler_params=pltpu.CompilerParams(needs_layout_passes=False),
            scratch_types=[
                pltpu.VMEM((_NPAD,), jnp.float32),
                pltpu.VMEM((_NPAD,), jnp.float32),
                pltpu.VMEM((_NPAD,), jnp.float32),
                pltpu.VMEM((_NPAD,), jnp.float32),
                pltpu.VMEM((_G * _NSF,), jnp.float32),
                pltpu.VMEM((_G * 8,), jnp.float32),
                pltpu.VMEM((_G * _K,), jnp.float32),
                pltpu.VMEM((_NSF + 16,), jnp.int32),
                pltpu.VMEM((_POOL + 16,), jnp.float32),
                pltpu.VMEM((_POOL + 16,), jnp.int32),
            ],
        )
        outs.append(sc(sm.reshape(-1), t0.reshape(-1), kxt, kyt, kzt, k2))
    return jnp.concatenate(outs).reshape(_NPAD, _K)[:n]
